# baseline (speedup 1.0000x reference)
import functools

import jax
import jax.numpy as jnp
from jax import lax
from jax.experimental import pallas as pl
from jax.experimental.pallas import tpu as pltpu
from jax.experimental.pallas import tpu_sc as plsc

_T, _D, _H, _E = 2048, 1024, 512, 64
_BM = 256
_NC = _T // _BM
_NS = _NC + _E - 1
_NW = 32
_CHUNK = _T // _NW


def _route_body(x_ref, wg_ref, slot_ref, pk_ref):
    x = x_ref[...]
    wg = wg_ref[...]
    logits = jnp.dot(x, wg, preferred_element_type=jnp.float32)
    lane = lax.broadcasted_iota(jnp.int32, (_T, _E), 1)
    m = jnp.max(logits, axis=1, keepdims=True)
    eid = jnp.min(jnp.where(logits == m, lane, _E), axis=1, keepdims=True)
    oh = (lane == eid).astype(jnp.float32)
    counts = jnp.sum(oh, axis=0, keepdims=True)
    jj = lax.broadcasted_iota(jnp.int32, (_E, 128), 0)
    ll = lax.broadcasted_iota(jnp.int32, (_E, 128), 1)
    lt = (jj < ll).astype(jnp.float32)
    cq = jnp.floor(counts * (1.0 / 256.0))
    cr_ = counts - 256.0 * cq
    off128 = (256.0 * jnp.dot(cq, lt, preferred_element_type=jnp.float32)
              + jnp.dot(cr_, lt, preferred_element_type=jnp.float32))
    off_e = off128[:, :_E]
    ri = lax.broadcasted_iota(jnp.int32, (128, 128), 0)
    ci = lax.broadcasted_iota(jnp.int32, (128, 128), 1)
    ls = (ci < ri).astype(jnp.float32)
    prefix = jnp.zeros((1, _E), jnp.float32)
    for c in range(_T // 128):
        ohc = oh[c * 128:(c + 1) * 128, :]
        rk = jnp.dot(ls, ohc, preferred_element_type=jnp.float32) + prefix
        slot_rows = jnp.sum(ohc * (rk + off_e), axis=1, keepdims=True)
        slot_ref[c * 128:(c + 1) * 128, :] = slot_rows.astype(jnp.int32)
        prefix = prefix + jnp.sum(ohc, axis=0, keepdims=True)

    pk_ref[...] = off128.astype(jnp.int32)


def _route(x, w_gate):
    return pl.pallas_call(
        _route_body,
        out_shape=[
            jax.ShapeDtypeStruct((_T, 1), jnp.int32),
            jax.ShapeDtypeStruct((1, 128), jnp.int32),
        ],
    )(x, w_gate)


_EPG = 4


def _gmm_body(pk_ref, xs_ref, w1_ref, w2_ref, out_ref):
    g = pl.program_id(0)

    @pl.when(g == 0)
    def _():
        out_ref[...] = jnp.zeros_like(out_ref)

    for k in range(_EPG):
        e = g * _EPG + k
        start = pk_ref[0, e]
        end = pk_ref[0, e + 1]
        s0 = (start // _BM) * _BM
        nch = (end - s0 + _BM - 1) // _BM

        def body(j, carry, k=k, start=start, end=end, s0=s0):
            s = pl.multiple_of(s0 + j * _BM, _BM)
            rows = xs_ref[pl.ds(s, _BM), :]
            ids = s + lax.broadcasted_iota(jnp.int32, (_BM, 1), 0)
            msk = ((ids >= start) & (ids < end)).astype(jnp.float32)
            h1 = jnp.dot(rows, w1_ref[k], preferred_element_type=jnp.float32)
            h1 = h1 * msk
            o = jnp.dot(h1, w2_ref[k], preferred_element_type=jnp.float32)
            out_ref[pl.ds(s, _BM), :] += o
            return carry

        lax.fori_loop(0, nch, body, 0)


def _gmm(pk, xs, w1, w2):
    grid_spec = pltpu.PrefetchScalarGridSpec(
        num_scalar_prefetch=1,
        grid=(_E // _EPG,),
        in_specs=[
            pl.BlockSpec((_T, _D), lambda g, p: (0, 0)),
            pl.BlockSpec((_EPG, _D, _H), lambda g, p: (g, 0, 0)),
            pl.BlockSpec((_EPG, _H, _D), lambda g, p: (g, 0, 0)),
        ],
        out_specs=pl.BlockSpec((_T, _D), lambda g, p: (0, 0)),
    )
    return pl.pallas_call(
        _gmm_body,
        grid_spec=grid_spec,
        out_shape=jax.ShapeDtypeStruct((_T, _D), jnp.float32),
        compiler_params=pltpu.CompilerParams(
            dimension_semantics=("arbitrary",)),
    )(pk, xs, w1, w2)


_HC = _CHUNK // 2


@functools.lru_cache(maxsize=None)
def _sc_kernels():
    mesh = plsc.VectorSubcoreMesh(core_axis_name="c", subcore_axis_name="s")
    deco = functools.partial(
        pl.kernel,
        mesh=mesh,
        out_type=jax.ShapeDtypeStruct((_T, _D), jnp.float32),
        scratch_types=[
            pltpu.VMEM((_HC,), jnp.int32),
            pltpu.VMEM((_HC,), jnp.int32),
            pltpu.VMEM((_HC, _D), jnp.float32),
            pltpu.VMEM((_HC, _D), jnp.float32),
            pltpu.SemaphoreType.DMA,
            pltpu.SemaphoreType.DMA,
        ],
    )

    @deco
    def dispatch(slot_hbm, x_hbm, out_hbm, idx0, idx1, r0, r1, s0, s1):
        wid = lax.axis_index("s") * 2 + lax.axis_index("c")
        base = wid * _CHUNK
        pltpu.sync_copy(slot_hbm.at[pl.ds(base, _HC)], idx0)
        pltpu.sync_copy(slot_hbm.at[pl.ds(base + _HC, _HC)], idx1)
        c0 = pltpu.async_copy(x_hbm.at[pl.ds(base, _HC)], r0, s0)
        c1 = pltpu.async_copy(x_hbm.at[pl.ds(base + _HC, _HC)], r1, s1)
        c0.wait()
        w0 = pltpu.async_copy(r0, out_hbm.at[idx0], s0)
        c1.wait()
        w1 = pltpu.async_copy(r1, out_hbm.at[idx1], s1)
        w0.wait()
        w1.wait()

    @deco
    def combine(slot_hbm, src_hbm, y_hbm, idx0, idx1, r0, r1, s0, s1):
        wid = lax.axis_index("s") * 2 + lax.axis_index("c")
        base = wid * _CHUNK
        pltpu.sync_copy(slot_hbm.at[pl.ds(base, _HC)], idx0)
        pltpu.sync_copy(slot_hbm.at[pl.ds(base + _HC, _HC)], idx1)
        g0 = pltpu.async_copy(src_hbm.at[idx0], r0, s0)
        g1 = pltpu.async_copy(src_hbm.at[idx1], r1, s1)
        g0.wait()
        o0 = pltpu.async_copy(r0, y_hbm.at[pl.ds(base, _HC)], s0)
        g1.wait()
        o1 = pltpu.async_copy(r1, y_hbm.at[pl.ds(base + _HC, _HC)], s1)
        o0.wait()
        o1.wait()

    return dispatch, combine


def kernel(x, w_gate, w1, w2):
    slot2d, pk = _route(x, w_gate)
    slot = slot2d.reshape(_T)
    dispatch, combine = _sc_kernels()
    xs = dispatch(slot, x)
    out_sorted = _gmm(pk, xs, w1, w2)
    return combine(slot, out_sorted)


# speedup vs baseline: 1.0302x; 1.0302x over previous
import functools

import jax
import jax.numpy as jnp
from jax import lax
from jax.experimental import pallas as pl
from jax.experimental.pallas import tpu as pltpu
from jax.experimental.pallas import tpu_sc as plsc

_T, _D, _H, _E = 2048, 1024, 512, 64
_BM = 128
_NC = _T // _BM
_NS = _NC + _E - 1
_NW = 32
_CHUNK = _T // _NW


def _route_body(x_ref, wg_ref, slot_ref, pk_ref):
    x = x_ref[...]
    wg = wg_ref[...]
    logits = jnp.dot(x, wg, preferred_element_type=jnp.float32)
    lane = lax.broadcasted_iota(jnp.int32, (_T, _E), 1)
    m = jnp.max(logits, axis=1, keepdims=True)
    eid = jnp.min(jnp.where(logits == m, lane, _E), axis=1, keepdims=True)
    oh = (lane == eid).astype(jnp.float32)
    counts = jnp.sum(oh, axis=0, keepdims=True)
    jj = lax.broadcasted_iota(jnp.int32, (_E, 128), 0)
    ll = lax.broadcasted_iota(jnp.int32, (_E, 128), 1)
    lt = (jj < ll).astype(jnp.float32)
    cq = jnp.floor(counts * (1.0 / 256.0))
    cr_ = counts - 256.0 * cq
    off128 = (256.0 * jnp.dot(cq, lt, preferred_element_type=jnp.float32)
              + jnp.dot(cr_, lt, preferred_element_type=jnp.float32))
    off_e = off128[:, :_E]
    ri = lax.broadcasted_iota(jnp.int32, (128, 128), 0)
    ci = lax.broadcasted_iota(jnp.int32, (128, 128), 1)
    ls = (ci < ri).astype(jnp.float32)
    prefix = jnp.zeros((1, _E), jnp.float32)
    for c in range(_T // 128):
        ohc = oh[c * 128:(c + 1) * 128, :]
        rk = jnp.dot(ls, ohc, preferred_element_type=jnp.float32) + prefix
        slot_rows = jnp.sum(ohc * (rk + off_e), axis=1, keepdims=True)
        slot_ref[c * 128:(c + 1) * 128, :] = slot_rows.astype(jnp.int32)
        prefix = prefix + jnp.sum(ohc, axis=0, keepdims=True)

    pk_ref[...] = off128.astype(jnp.int32)


def _route(x, w_gate):
    return pl.pallas_call(
        _route_body,
        out_shape=[
            jax.ShapeDtypeStruct((_T, 1), jnp.int32),
            jax.ShapeDtypeStruct((1, 128), jnp.int32),
        ],
    )(x, w_gate)


_EPG = 4


def _gmm_body(pk_ref, xs_ref, w1_ref, w2_ref, out_ref):
    g = pl.program_id(0)

    @pl.when(g == 0)
    def _():
        out_ref[...] = jnp.zeros_like(out_ref)

    for k in range(_EPG):
        e = g * _EPG + k
        start = pk_ref[e]
        end = pk_ref[e + 1]
        s0 = (start // _BM) * _BM
        nch = (end - s0 + _BM - 1) // _BM

        def body(j, carry, k=k, start=start, end=end, s0=s0):
            s = pl.multiple_of(s0 + j * _BM, _BM)
            rows = xs_ref[pl.ds(s, _BM), :]
            ids = s + lax.broadcasted_iota(jnp.int32, (_BM, 1), 0)
            msk = ((ids >= start) & (ids < end)).astype(jnp.float32)
            h1 = jnp.dot(rows, w1_ref[k], preferred_element_type=jnp.float32)
            h1 = h1 * msk
            o = jnp.dot(h1, w2_ref[k], preferred_element_type=jnp.float32)
            out_ref[pl.ds(s, _BM), :] += o
            return carry

        lax.fori_loop(0, nch, body, 0)


def _gmm(pk, xs, w1, w2):
    grid_spec = pltpu.PrefetchScalarGridSpec(
        num_scalar_prefetch=1,
        grid=(_E // _EPG,),
        in_specs=[
            pl.BlockSpec((_T, _D), lambda g, p: (0, 0)),
            pl.BlockSpec((_EPG, _D, _H), lambda g, p: (g, 0, 0)),
            pl.BlockSpec((_EPG, _H, _D), lambda g, p: (g, 0, 0)),
        ],
        out_specs=pl.BlockSpec((_T, _D), lambda g, p: (0, 0)),
    )
    return pl.pallas_call(
        _gmm_body,
        grid_spec=grid_spec,
        out_shape=jax.ShapeDtypeStruct((_T, _D), jnp.float32),
        compiler_params=pltpu.CompilerParams(
            dimension_semantics=("arbitrary",)),
    )(pk, xs, w1, w2)


_HC = _CHUNK // 2


@functools.lru_cache(maxsize=None)
def _sc_kernels():
    mesh = plsc.VectorSubcoreMesh(core_axis_name="c", subcore_axis_name="s")
    deco = functools.partial(
        pl.kernel,
        mesh=mesh,
        out_type=jax.ShapeDtypeStruct((_T, _D), jnp.float32),
        scratch_types=[
            pltpu.VMEM((_HC,), jnp.int32),
            pltpu.VMEM((_HC,), jnp.int32),
            pltpu.VMEM((_HC, _D), jnp.float32),
            pltpu.VMEM((_HC, _D), jnp.float32),
            pltpu.SemaphoreType.DMA,
            pltpu.SemaphoreType.DMA,
        ],
    )

    @deco
    def dispatch(slot_hbm, x_hbm, out_hbm, idx0, idx1, r0, r1, s0, s1):
        wid = lax.axis_index("s") * 2 + lax.axis_index("c")
        base = wid * _CHUNK
        pltpu.sync_copy(slot_hbm.at[pl.ds(base, _HC)], idx0)
        pltpu.sync_copy(slot_hbm.at[pl.ds(base + _HC, _HC)], idx1)
        c0 = pltpu.async_copy(x_hbm.at[pl.ds(base, _HC)], r0, s0)
        c1 = pltpu.async_copy(x_hbm.at[pl.ds(base + _HC, _HC)], r1, s1)
        c0.wait()
        w0 = pltpu.async_copy(r0, out_hbm.at[idx0], s0)
        c1.wait()
        w1 = pltpu.async_copy(r1, out_hbm.at[idx1], s1)
        w0.wait()
        w1.wait()

    @deco
    def combine(slot_hbm, src_hbm, y_hbm, idx0, idx1, r0, r1, s0, s1):
        wid = lax.axis_index("s") * 2 + lax.axis_index("c")
        base = wid * _CHUNK
        pltpu.sync_copy(slot_hbm.at[pl.ds(base, _HC)], idx0)
        pltpu.sync_copy(slot_hbm.at[pl.ds(base + _HC, _HC)], idx1)
        g0 = pltpu.async_copy(src_hbm.at[idx0], r0, s0)
        g1 = pltpu.async_copy(src_hbm.at[idx1], r1, s1)
        g0.wait()
        o0 = pltpu.async_copy(r0, y_hbm.at[pl.ds(base, _HC)], s0)
        g1.wait()
        o1 = pltpu.async_copy(r1, y_hbm.at[pl.ds(base + _HC, _HC)], s1)
        o0.wait()
        o1.wait()

    return dispatch, combine


def kernel(x, w_gate, w1, w2):
    slot2d, pk = _route(x, w_gate)
    slot = slot2d.reshape(_T)
    dispatch, combine = _sc_kernels()
    xs = dispatch(slot, x)
    out_sorted = _gmm(pk.reshape(128), xs, w1, w2)
    return combine(slot, out_sorted)


# speedup vs baseline: 1.0419x; 1.0113x over previous
import functools

import jax
import jax.numpy as jnp
from jax import lax
from jax.experimental import pallas as pl
from jax.experimental.pallas import tpu as pltpu
from jax.experimental.pallas import tpu_sc as plsc

_T, _D, _H, _E = 2048, 1024, 512, 64
_BM = 128
_NC = _T // _BM
_NS = _NC + _E - 1
_NW = 32
_CHUNK = _T // _NW


def _route_body(x_ref, wg_ref, slot_ref, pk_ref):
    x = x_ref[...]
    wg = wg_ref[...]
    logits = jnp.dot(x, wg, preferred_element_type=jnp.float32)
    lane = lax.broadcasted_iota(jnp.int32, (_T, _E), 1)
    m = jnp.max(logits, axis=1, keepdims=True)
    eid = jnp.min(jnp.where(logits == m, lane, _E), axis=1, keepdims=True)
    oh = (lane == eid).astype(jnp.float32)
    counts = jnp.sum(oh, axis=0, keepdims=True)
    jj = lax.broadcasted_iota(jnp.int32, (_E, 128), 0)
    ll = lax.broadcasted_iota(jnp.int32, (_E, 128), 1)
    lt = (jj < ll).astype(jnp.float32)
    cq = jnp.floor(counts * (1.0 / 256.0))
    cr_ = counts - 256.0 * cq
    off128 = (256.0 * jnp.dot(cq, lt, preferred_element_type=jnp.float32)
              + jnp.dot(cr_, lt, preferred_element_type=jnp.float32))
    off_e = off128[:, :_E]
    ri = lax.broadcasted_iota(jnp.int32, (128, 128), 0)
    ci = lax.broadcasted_iota(jnp.int32, (128, 128), 1)
    ls = (ci < ri).astype(jnp.float32)
    prefix = jnp.zeros((1, _E), jnp.float32)
    for c in range(_T // 128):
        ohc = oh[c * 128:(c + 1) * 128, :]
        rk = jnp.dot(ls, ohc, preferred_element_type=jnp.float32) + prefix
        slot_rows = jnp.sum(ohc * (rk + off_e), axis=1, keepdims=True)
        slot_ref[c * 128:(c + 1) * 128, :] = slot_rows.astype(jnp.int32)
        prefix = prefix + jnp.sum(ohc, axis=0, keepdims=True)

    pk_ref[...] = off128.astype(jnp.int32)


def _route(x, w_gate):
    return pl.pallas_call(
        _route_body,
        out_shape=[
            jax.ShapeDtypeStruct((_T, 1), jnp.int32),
            jax.ShapeDtypeStruct((1, 128), jnp.int32),
        ],
    )(x, w_gate)


_EPG = 4


def _gmm_body(pk_ref, xs_ref, w1_ref, w2_ref, out_ref):
    for k in range(_EPG):
        e = pl.program_id(0) * _EPG + k
        start = pk_ref[e]
        end = pk_ref[e + 1]
        s0 = (start // _BM) * _BM
        nch = (end - s0 + _BM - 1) // _BM

        def body(j, carry, k=k, start=start, end=end, s0=s0):
            s = pl.multiple_of(s0 + j * _BM, _BM)
            rows = xs_ref[pl.ds(s, _BM), :]
            ids = s + lax.broadcasted_iota(jnp.int32, (_BM, 1), 0)
            msk = ((ids >= start) & (ids < end)).astype(jnp.float32)
            h1 = jnp.dot(rows, w1_ref[k], preferred_element_type=jnp.float32)
            h1 = h1 * msk
            o = jnp.dot(h1, w2_ref[k], preferred_element_type=jnp.float32)
            fresh = (j > 0) | (start == s0)

            @pl.when(fresh)
            def _():
                out_ref[pl.ds(s, _BM), :] = o

            @pl.when(jnp.logical_not(fresh))
            def _():
                out_ref[pl.ds(s, _BM), :] += o

            return carry

        lax.fori_loop(0, nch, body, 0)


def _gmm(pk, xs, w1, w2):
    grid_spec = pltpu.PrefetchScalarGridSpec(
        num_scalar_prefetch=1,
        grid=(_E // _EPG,),
        in_specs=[
            pl.BlockSpec((_T, _D), lambda g, p: (0, 0)),
            pl.BlockSpec((_EPG, _D, _H), lambda g, p: (g, 0, 0)),
            pl.BlockSpec((_EPG, _H, _D), lambda g, p: (g, 0, 0)),
        ],
        out_specs=pl.BlockSpec((_T, _D), lambda g, p: (0, 0)),
    )
    return pl.pallas_call(
        _gmm_body,
        grid_spec=grid_spec,
        out_shape=jax.ShapeDtypeStruct((_T, _D), jnp.float32),
        compiler_params=pltpu.CompilerParams(
            dimension_semantics=("arbitrary",)),
    )(pk, xs, w1, w2)


_HC = _CHUNK // 2


@functools.lru_cache(maxsize=None)
def _sc_kernels():
    mesh = plsc.VectorSubcoreMesh(core_axis_name="c", subcore_axis_name="s")
    deco = functools.partial(
        pl.kernel,
        mesh=mesh,
        out_type=jax.ShapeDtypeStruct((_T, _D), jnp.float32),
        scratch_types=[
            pltpu.VMEM((_HC,), jnp.int32),
            pltpu.VMEM((_HC,), jnp.int32),
            pltpu.VMEM((_HC, _D), jnp.float32),
            pltpu.VMEM((_HC, _D), jnp.float32),
            pltpu.SemaphoreType.DMA,
            pltpu.SemaphoreType.DMA,
        ],
    )

    @deco
    def dispatch(slot_hbm, x_hbm, out_hbm, idx0, idx1, r0, r1, s0, s1):
        wid = lax.axis_index("s") * 2 + lax.axis_index("c")
        base = wid * _CHUNK
        pltpu.sync_copy(slot_hbm.at[pl.ds(base, _HC)], idx0)
        pltpu.sync_copy(slot_hbm.at[pl.ds(base + _HC, _HC)], idx1)
        c0 = pltpu.async_copy(x_hbm.at[pl.ds(base, _HC)], r0, s0)
        c1 = pltpu.async_copy(x_hbm.at[pl.ds(base + _HC, _HC)], r1, s1)
        c0.wait()
        w0 = pltpu.async_copy(r0, out_hbm.at[idx0], s0)
        c1.wait()
        w1 = pltpu.async_copy(r1, out_hbm.at[idx1], s1)
        w0.wait()
        w1.wait()

    @deco
    def combine(slot_hbm, src_hbm, y_hbm, idx0, idx1, r0, r1, s0, s1):
        wid = lax.axis_index("s") * 2 + lax.axis_index("c")
        base = wid * _CHUNK
        pltpu.sync_copy(slot_hbm.at[pl.ds(base, _HC)], idx0)
        pltpu.sync_copy(slot_hbm.at[pl.ds(base + _HC, _HC)], idx1)
        g0 = pltpu.async_copy(src_hbm.at[idx0], r0, s0)
        g1 = pltpu.async_copy(src_hbm.at[idx1], r1, s1)
        g0.wait()
        o0 = pltpu.async_copy(r0, y_hbm.at[pl.ds(base, _HC)], s0)
        g1.wait()
        o1 = pltpu.async_copy(r1, y_hbm.at[pl.ds(base + _HC, _HC)], s1)
        o0.wait()
        o1.wait()

    return dispatch, combine


def kernel(x, w_gate, w1, w2):
    slot2d, pk = _route(x, w_gate)
    slot = slot2d.reshape(_T)
    dispatch, combine = _sc_kernels()
    xs = dispatch(slot, x)
    out_sorted = _gmm(pk.reshape(128), xs, w1, w2)
    return combine(slot, out_sorted)


# speedup vs baseline: 1.0428x; 1.0008x over previous
import functools

import jax
import jax.numpy as jnp
from jax import lax
from jax.experimental import pallas as pl
from jax.experimental.pallas import tpu as pltpu
from jax.experimental.pallas import tpu_sc as plsc

_T, _D, _H, _E = 2048, 1024, 512, 64
_BM = 128
_NC = _T // _BM
_NS = _NC + _E - 1
_NW = 32
_CHUNK = _T // _NW


def _route_body(x_ref, wg_ref, slot_ref, pk_ref):
    x = x_ref[...]
    wg = wg_ref[...]
    logits = jnp.dot(x, wg, preferred_element_type=jnp.float32)
    lane = lax.broadcasted_iota(jnp.int32, (_T, _E), 1)
    m = jnp.max(logits, axis=1, keepdims=True)
    eid = jnp.min(jnp.where(logits == m, lane, _E), axis=1, keepdims=True)
    oh = (lane == eid).astype(jnp.float32)
    counts = jnp.sum(oh, axis=0, keepdims=True)
    jj = lax.broadcasted_iota(jnp.int32, (_E, 128), 0)
    ll = lax.broadcasted_iota(jnp.int32, (_E, 128), 1)
    lt = (jj < ll).astype(jnp.float32)
    cq = jnp.floor(counts * (1.0 / 256.0))
    cr_ = counts - 256.0 * cq
    off128 = (256.0 * jnp.dot(cq, lt, preferred_element_type=jnp.float32)
              + jnp.dot(cr_, lt, preferred_element_type=jnp.float32))
    off_e = off128[:, :_E]
    ri = lax.broadcasted_iota(jnp.int32, (128, 128), 0)
    ci = lax.broadcasted_iota(jnp.int32, (128, 128), 1)
    ls = (ci < ri).astype(jnp.float32)
    prefix = jnp.zeros((1, _E), jnp.float32)
    for c in range(_T // 128):
        ohc = oh[c * 128:(c + 1) * 128, :]
        rk = jnp.dot(ls, ohc, preferred_element_type=jnp.float32) + prefix
        slot_rows = jnp.sum(ohc * (rk + off_e), axis=1, keepdims=True)
        slot_ref[c * 128:(c + 1) * 128, :] = slot_rows.astype(jnp.int32)
        prefix = prefix + jnp.sum(ohc, axis=0, keepdims=True)

    pk_ref[...] = off128.astype(jnp.int32)


def _route(x, w_gate):
    return pl.pallas_call(
        _route_body,
        out_shape=[
            jax.ShapeDtypeStruct((_T, 1), jnp.int32),
            jax.ShapeDtypeStruct((1, 128), jnp.int32),
        ],
    )(x, w_gate)


_EPG = 4


def _gmm_body(pk_ref, xs_ref, w1_ref, w2_ref, out_ref):
    for k in range(_EPG):
        e = pl.program_id(0) * _EPG + k
        start = pk_ref[e]
        end = pk_ref[e + 1]
        s0 = (start // _BM) * _BM
        nch = (end - s0 + _BM - 1) // _BM

        def body(j, carry, k=k, start=start, end=end, s0=s0):
            s = pl.multiple_of(s0 + j * _BM, _BM)
            rows = xs_ref[pl.ds(s, _BM), :]
            ids = s + lax.broadcasted_iota(jnp.int32, (_BM, 1), 0)
            msk = ((ids >= start) & (ids < end)).astype(jnp.float32)
            h1 = jnp.dot(rows, w1_ref[k], preferred_element_type=jnp.float32)
            h1 = h1 * msk
            o = jnp.dot(h1, w2_ref[k], preferred_element_type=jnp.float32)
            fresh = (j > 0) | (start == s0)

            @pl.when(fresh)
            def _():
                out_ref[pl.ds(s, _BM), :] = o

            @pl.when(jnp.logical_not(fresh))
            def _():
                out_ref[pl.ds(s, _BM), :] += o

            return carry

        lax.fori_loop(0, nch, body, 0)


def _gmm(pk, xs, w1, w2):
    return pl.pallas_call(
        _gmm_body,
        grid=(_E // _EPG,),
        in_specs=[
            pl.BlockSpec(memory_space=pltpu.SMEM),
            pl.BlockSpec((_T, _D), lambda g: (0, 0)),
            pl.BlockSpec((_EPG, _D, _H), lambda g: (g, 0, 0)),
            pl.BlockSpec((_EPG, _H, _D), lambda g: (g, 0, 0)),
        ],
        out_specs=pl.BlockSpec((_T, _D), lambda g: (0, 0)),
        out_shape=jax.ShapeDtypeStruct((_T, _D), jnp.float32),
        compiler_params=pltpu.CompilerParams(
            dimension_semantics=("arbitrary",)),
    )(pk, xs, w1, w2)


_HC = _CHUNK // 2


@functools.lru_cache(maxsize=None)
def _sc_kernels():
    mesh = plsc.VectorSubcoreMesh(core_axis_name="c", subcore_axis_name="s")
    deco = functools.partial(
        pl.kernel,
        mesh=mesh,
        out_type=jax.ShapeDtypeStruct((_T, _D), jnp.float32),
        scratch_types=[
            pltpu.VMEM((_HC,), jnp.int32),
            pltpu.VMEM((_HC,), jnp.int32),
            pltpu.VMEM((_HC, _D), jnp.float32),
            pltpu.VMEM((_HC, _D), jnp.float32),
            pltpu.SemaphoreType.DMA,
            pltpu.SemaphoreType.DMA,
        ],
    )

    @deco
    def dispatch(slot_hbm, x_hbm, out_hbm, idx0, idx1, r0, r1, s0, s1):
        wid = lax.axis_index("s") * 2 + lax.axis_index("c")
        base = wid * _CHUNK
        pltpu.sync_copy(slot_hbm.at[pl.ds(base, _HC)], idx0)
        pltpu.sync_copy(slot_hbm.at[pl.ds(base + _HC, _HC)], idx1)
        c0 = pltpu.async_copy(x_hbm.at[pl.ds(base, _HC)], r0, s0)
        c1 = pltpu.async_copy(x_hbm.at[pl.ds(base + _HC, _HC)], r1, s1)
        c0.wait()
        w0 = pltpu.async_copy(r0, out_hbm.at[idx0], s0)
        c1.wait()
        w1 = pltpu.async_copy(r1, out_hbm.at[idx1], s1)
        w0.wait()
        w1.wait()

    @deco
    def combine(slot_hbm, src_hbm, y_hbm, idx0, idx1, r0, r1, s0, s1):
        wid = lax.axis_index("s") * 2 + lax.axis_index("c")
        base = wid * _CHUNK
        pltpu.sync_copy(slot_hbm.at[pl.ds(base, _HC)], idx0)
        pltpu.sync_copy(slot_hbm.at[pl.ds(base + _HC, _HC)], idx1)
        g0 = pltpu.async_copy(src_hbm.at[idx0], r0, s0)
        g1 = pltpu.async_copy(src_hbm.at[idx1], r1, s1)
        g0.wait()
        o0 = pltpu.async_copy(r0, y_hbm.at[pl.ds(base, _HC)], s0)
        g1.wait()
        o1 = pltpu.async_copy(r1, y_hbm.at[pl.ds(base + _HC, _HC)], s1)
        o0.wait()
        o1.wait()

    return dispatch, combine


def kernel(x, w_gate, w1, w2):
    slot2d, pk = _route(x, w_gate)
    slot = slot2d.reshape(_T)
    dispatch, combine = _sc_kernels()
    xs = dispatch(slot, x)
    out_sorted = _gmm(pk.reshape(128), xs, w1, w2)
    return combine(slot, out_sorted)
